# baseline (speedup 1.0000x reference)
import functools

import jax
import jax.numpy as jnp
from jax import lax
from jax.experimental import pallas as pl
from jax.experimental.pallas import tpu as pltpu
from jax.experimental.pallas import tpu_sc as plsc

VOCAB = 1000
VPAD = 1024
NB = VPAD // 128
TAILV = VOCAB - 7 * 128
NC, NS, L = 2, 16, 16
NW = NC * NS
B, T = 16, 2048
N_TOK = B * T
TPW = N_TOK // NW
G = 128
NG = TPW // G
NSTEP = NG * NB


def _lse_body(table_ref, lse_ref):
    x = table_ref[...]
    m = jnp.max(x, axis=1, keepdims=True)
    s = jnp.sum(jnp.exp(x - m), axis=1, keepdims=True)
    lse_ref[...] = m + jnp.log(s)


def _lse(table):
    lse2d = pl.pallas_call(
        _lse_body,
        out_shape=jax.ShapeDtypeStruct((VOCAB, 1), jnp.float32),
    )(table)
    return lse2d.reshape(VOCAB)


def _sc_body(idx_hbm, tgt_hbm, table_hbm, tail_hbm, out_hbm, loss_hbm,
             hist_hbm, idx_v, tgt_v, hist_v, g_a, g_b, t_a, t_b,
             ix_a, ix_b, acc_v, table_sp,
             sem_ga, sem_gb, sem_sa, sem_sb):
    wid = lax.axis_index("c") * NS + lax.axis_index("s")
    sid = lax.axis_index("s")
    base = wid * TPW
    bidx = wid >> 1
    tcol0 = (wid & 1) * TPW
    pltpu.sync_copy(idx_hbm.at[pl.ds(base, TPW)], idx_v)
    pltpu.sync_copy(tgt_hbm.at[pl.ds(base, TPW)], tgt_v)
    lane = lax.iota(jnp.int32, L)
    fzero = jnp.zeros((L,), jnp.float32)
    fone = fzero + 1.0
    for k in range(VPAD // L):
        hist_v[pl.ds(L * k, L)] = fzero
    rowk = [lane + L * k for k in range(G // L)]

    gbuf = (g_a, g_b)
    tbuf = (t_a, t_b)
    ixbuf = (ix_a, ix_b)
    gsem = (sem_ga, sem_gb)
    ssem = (sem_sa, sem_sb)
    diags = [(lane + d) & (L - 1) for d in range(L)]

    def build_idx(g2, p2, par):
        for k in range(G // L):
            iv = idx_v[pl.ds(g2 * G + L * k, L)]
            if p2 == NB - 1:
                ixbuf[par][pl.ds(L * k, L)] = iv
            else:
                ixbuf[par][pl.ds(L * k, L)] = iv + (p2 * VOCAB)

    def gather_start(p2, par):
        src = tail_hbm if p2 == NB - 1 else table_sp
        pltpu.async_copy(src.at[ixbuf[par]], gbuf[par], gsem[par])

    def gather_wait(p2, par):
        src = tail_hbm if p2 == NB - 1 else table_sp
        pltpu.make_async_copy(
            src.at[ixbuf[par]], gbuf[par], gsem[par]).wait()

    def store_dsts(g, p, par):
        vn = TAILV if p == NB - 1 else 128
        src = tbuf[par].at[pl.ds(0, TAILV)] if p == NB - 1 else tbuf[par]
        dst = out_hbm.at[bidx, pl.ds(128 * p, vn), pl.ds(tcol0 + g * G, G)]
        return src, dst

    def store_start(g, p, par):
        src, dst = store_dsts(g, p, par)
        pltpu.async_copy(src, dst, ssem[par])

    def store_drain(g, p, par):
        src, dst = store_dsts(g, p, par)
        pltpu.make_async_copy(src, dst, ssem[par]).wait()

    def transpose(p, par):
        gb, tb = gbuf[par], tbuf[p & 1]

        ntile = 7 * (G // L) if p == NB - 1 else 8 * (G // L)

        @plsc.parallel_loop(0, ntile, unroll=4)
        def _(tau):
            w0 = (tau >> 3) << 4
            t0 = (tau & 7) << 4
            row = lane + t0
            for d in range(L):
                col = diags[d] + w0
                v = plsc.load_gather(gb, [row, col])
                plsc.store_scatter(tb, [col, row], v)

    def loss_step(g, p, par, acc):
        gb = gbuf[par]
        for k in range(G // L):
            off = g * G + L * k
            tgs = tgt_v[pl.ds(off, L)]
            if p == 0:
                ids = idx_v[pl.ds(off, L)]
                plsc.addupdate_scatter(hist_v, [ids], fone)
            tl = plsc.load_gather(gb, [rowk[k], tgs & 127])
            acc = acc + jnp.where((tgs >> 7) == p, tl, 0.0)
        return acc

    for pp in range(7):
        @pl.when(sid < 15)
        def _():
            pltpu.sync_copy(
                table_hbm.at[pl.ds(sid * 64, 64), pl.ds(128 * pp, 128)],
                table_sp.at[pl.ds(pp * VOCAB + sid * 64, 64)])

        @pl.when(sid == 15)
        def _():
            pltpu.sync_copy(
                table_hbm.at[pl.ds(960, 40), pl.ds(128 * pp, 128)],
                table_sp.at[pl.ds(pp * VOCAB + 960, 40)])
    plsc.subcore_barrier()

    for p0 in range(2):
        build_idx(0, p0, p0)
        gather_start(p0, p0)

    def body(g, acc):
        for p in range(NB):
            par = p & 1
            spar = p & 1
            gather_wait(p, par)

            @pl.when(g * NB + p >= 2)
            def _():
                store_drain(g if p >= 2 else g - 1, (p - 2) % NB, spar)

            transpose(p, par)
            store_start(g, p, spar)
            acc = loss_step(g, p, par, acc)

            g2 = g + 1 if p >= NB - 2 else g
            p2 = (p + 2) % NB

            @pl.when(g2 < NG)
            def _():
                build_idx(g2, p2, par)
                gather_start(p2, par)
        return acc

    acc = lax.fori_loop(0, NG, body, jnp.zeros((L,), jnp.float32))
    store_drain(NG - 1, NB - 2, 0)
    store_drain(NG - 1, NB - 1, 1)
    acc_v[pl.ds(0, L)] = acc
    for k in range(1, 128 // L):
        acc_v[pl.ds(k * L, L)] = fzero
    pltpu.sync_copy(acc_v, loss_hbm.at[pl.ds(wid * 128, 128)])
    pltpu.sync_copy(hist_v, hist_hbm.at[pl.ds(wid * VPAD, VPAD)])


@functools.cache
def _sc_gather():
    return pl.kernel(
        _sc_body,
        out_type=(
            jax.ShapeDtypeStruct((B, VOCAB, T), jnp.float32),
            jax.ShapeDtypeStruct((NW * 128,), jnp.float32),
            jax.ShapeDtypeStruct((NW * VPAD,), jnp.float32),
        ),
        mesh=plsc.VectorSubcoreMesh(core_axis_name="c", subcore_axis_name="s"),
        compiler_params=pltpu.CompilerParams(needs_layout_passes=False),
        scratch_types=(
            pltpu.VMEM((TPW,), jnp.int32),
            pltpu.VMEM((TPW,), jnp.int32),
            pltpu.VMEM((VPAD,), jnp.float32),
            pltpu.VMEM((G, 128), jnp.float32),
            pltpu.VMEM((G, 128), jnp.float32),
            pltpu.VMEM((128, G), jnp.float32),
            pltpu.VMEM((128, G), jnp.float32),
            pltpu.VMEM((G,), jnp.int32),
            pltpu.VMEM((G,), jnp.int32),
            pltpu.VMEM((128,), jnp.float32),
            pltpu.VMEM_SHARED((VOCAB * 7, 128), jnp.float32),
            pltpu.SemaphoreType.DMA,
            pltpu.SemaphoreType.DMA,
            pltpu.SemaphoreType.DMA,
            pltpu.SemaphoreType.DMA,
        ),
    )


def kernel(idx, targets, table):
    idx_f = idx.reshape(-1)
    tgt_f = targets.reshape(-1)
    tail = jnp.pad(lax.slice(table, (0, 7 * 128), (VOCAB, VOCAB)),
                   ((0, 0), (0, VPAD - VOCAB)))
    lse = _lse(table)
    out_t, tgt_part, hist_part = _sc_gather()(idx_f, tgt_f, table, tail)
    logits = jnp.transpose(out_t, (0, 2, 1))
    hist = jnp.sum(hist_part.reshape(NW, VPAD), axis=0)[:VOCAB]
    loss = (jnp.dot(hist, lse) - jnp.sum(tgt_part)) / float(N_TOK)
    return (logits, loss)


# speedup vs baseline: 1.0716x; 1.0716x over previous
import functools

import jax
import jax.numpy as jnp
from jax import lax
from jax.experimental import pallas as pl
from jax.experimental.pallas import tpu as pltpu
from jax.experimental.pallas import tpu_sc as plsc

VOCAB = 1000
VPAD = 1024
NB = VPAD // 128
TAILV = VOCAB - 7 * 128
NC, NS, L = 2, 16, 16
NW = NC * NS
B, T = 16, 2048
N_TOK = B * T
TPW = N_TOK // NW
G = 128
NG = TPW // G
NSTEP = NG * NB


def _lse_body(table_ref, lse_ref):
    x = table_ref[...]
    m = jnp.max(x, axis=1, keepdims=True)
    s = jnp.sum(jnp.exp(x - m), axis=1, keepdims=True)
    lse_ref[...] = m + jnp.log(s)


def _lse(table):
    lse2d = pl.pallas_call(
        _lse_body,
        out_shape=jax.ShapeDtypeStruct((VOCAB, 1), jnp.float32),
    )(table)
    return lse2d.reshape(VOCAB)


def _sc_body(idx_hbm, tgt_hbm, table_hbm, tail_hbm, out_hbm, loss_hbm,
             hist_hbm, idx_v, tgt_v, hist_v, g_a, g_b, t_a, t_b,
             ix_a, ix_b, acc_v, table_sp,
             sem_ga, sem_gb, sem_sa, sem_sb):
    wid = lax.axis_index("c") * NS + lax.axis_index("s")
    sid = lax.axis_index("s")
    base = wid * TPW
    bidx = wid >> 1
    tcol0 = (wid & 1) * TPW
    pltpu.sync_copy(idx_hbm.at[pl.ds(base, TPW)], idx_v)
    pltpu.sync_copy(tgt_hbm.at[pl.ds(base, TPW)], tgt_v)
    lane = lax.iota(jnp.int32, L)
    fzero = jnp.zeros((L,), jnp.float32)
    fone = fzero + 1.0
    for k in range(VPAD // L):
        hist_v[pl.ds(L * k, L)] = fzero
    rowk = [lane + L * k for k in range(G // L)]

    gbuf = (g_a, g_b)
    tbuf = (t_a, t_b)
    ixbuf = (ix_a, ix_b)
    gsem = (sem_ga, sem_gb)
    ssem = (sem_sa, sem_sb)
    diags = [(lane + d) & (L - 1) for d in range(L)]

    def build_idx(g2, p2, par):
        for k in range(G // L):
            iv = idx_v[pl.ds(g2 * G + L * k, L)]
            if p2 == NB - 1:
                ixbuf[par][pl.ds(L * k, L)] = iv
            else:
                ixbuf[par][pl.ds(L * k, L)] = iv + (p2 * VOCAB)

    def gather_start(p2, par):
        src = tail_hbm if p2 == NB - 1 else table_sp
        pltpu.async_copy(src.at[ixbuf[par]], gbuf[par], gsem[par])

    def gather_wait(p2, par):
        src = tail_hbm if p2 == NB - 1 else table_sp
        pltpu.make_async_copy(
            src.at[ixbuf[par]], gbuf[par], gsem[par]).wait()

    def store_dsts(g, p, par):
        vn = TAILV if p == NB - 1 else 128
        src = tbuf[par].at[pl.ds(0, TAILV)] if p == NB - 1 else tbuf[par]
        dst = out_hbm.at[bidx, pl.ds(128 * p, vn), pl.ds(tcol0 + g * G, G)]
        return src, dst

    def store_start(g, p, par):
        src, dst = store_dsts(g, p, par)
        pltpu.async_copy(src, dst, ssem[par])

    def store_drain(g, p, par):
        src, dst = store_dsts(g, p, par)
        pltpu.make_async_copy(src, dst, ssem[par]).wait()

    def transpose(p, par):
        gb, tb = gbuf[par], tbuf[p & 1]

        @plsc.parallel_loop(0, (G // L) * (128 // L), unroll=4)
        def _(tau):
            t0 = (tau >> 3) << 4
            w0 = (tau & 7) << 4
            row = lane + t0
            for d in range(L):
                col = diags[d] + w0
                v = plsc.load_gather(gb, [row, col])
                plsc.store_scatter(tb, [col, row], v)

    def loss_step(g, p, par, acc):
        gb = gbuf[par]
        for k in range(G // L):
            off = g * G + L * k
            tgs = tgt_v[pl.ds(off, L)]
            if p == 0:
                ids = idx_v[pl.ds(off, L)]
                plsc.addupdate_scatter(hist_v, [ids], fone)
            tl = plsc.load_gather(gb, [rowk[k], tgs & 127])
            acc = acc + jnp.where((tgs >> 7) == p, tl, 0.0)
        return acc

    for pp in range(7):
        @pl.when(sid < 15)
        def _():
            pltpu.sync_copy(
                table_hbm.at[pl.ds(sid * 64, 64), pl.ds(128 * pp, 128)],
                table_sp.at[pl.ds(pp * VOCAB + sid * 64, 64)])

        @pl.when(sid == 15)
        def _():
            pltpu.sync_copy(
                table_hbm.at[pl.ds(960, 40), pl.ds(128 * pp, 128)],
                table_sp.at[pl.ds(pp * VOCAB + 960, 40)])
    plsc.subcore_barrier()

    for p0 in range(2):
        build_idx(0, p0, p0)
        gather_start(p0, p0)

    def body(g, acc):
        for p in range(NB):
            par = p & 1
            spar = p & 1
            gather_wait(p, par)

            @pl.when(g * NB + p >= 2)
            def _():
                store_drain(g if p >= 2 else g - 1, (p - 2) % NB, spar)

            transpose(p, par)
            store_start(g, p, spar)
            acc = loss_step(g, p, par, acc)

            g2 = g + 1 if p >= NB - 2 else g
            p2 = (p + 2) % NB

            @pl.when(g2 < NG)
            def _():
                build_idx(g2, p2, par)
                gather_start(p2, par)
        return acc

    acc = lax.fori_loop(0, NG, body, jnp.zeros((L,), jnp.float32))
    store_drain(NG - 1, NB - 2, 0)
    store_drain(NG - 1, NB - 1, 1)
    acc_v[pl.ds(0, L)] = acc
    for k in range(1, 128 // L):
        acc_v[pl.ds(k * L, L)] = fzero
    pltpu.sync_copy(acc_v, loss_hbm.at[pl.ds(wid * 128, 128)])
    pltpu.sync_copy(hist_v, hist_hbm.at[pl.ds(wid * VPAD, VPAD)])


@functools.cache
def _sc_gather():
    return pl.kernel(
        _sc_body,
        out_type=(
            jax.ShapeDtypeStruct((B, VOCAB, T), jnp.float32),
            jax.ShapeDtypeStruct((NW * 128,), jnp.float32),
            jax.ShapeDtypeStruct((NW * VPAD,), jnp.float32),
        ),
        mesh=plsc.VectorSubcoreMesh(core_axis_name="c", subcore_axis_name="s"),
        compiler_params=pltpu.CompilerParams(needs_layout_passes=False),
        scratch_types=(
            pltpu.VMEM((TPW,), jnp.int32),
            pltpu.VMEM((TPW,), jnp.int32),
            pltpu.VMEM((VPAD,), jnp.float32),
            pltpu.VMEM((G, 128), jnp.float32),
            pltpu.VMEM((G, 128), jnp.float32),
            pltpu.VMEM((128, G), jnp.float32),
            pltpu.VMEM((128, G), jnp.float32),
            pltpu.VMEM((G,), jnp.int32),
            pltpu.VMEM((G,), jnp.int32),
            pltpu.VMEM((128,), jnp.float32),
            pltpu.VMEM_SHARED((VOCAB * 7, 128), jnp.float32),
            pltpu.SemaphoreType.DMA,
            pltpu.SemaphoreType.DMA,
            pltpu.SemaphoreType.DMA,
            pltpu.SemaphoreType.DMA,
        ),
    )


def kernel(idx, targets, table):
    idx_f = idx.reshape(-1)
    tgt_f = targets.reshape(-1)
    tail = jnp.pad(lax.slice(table, (0, 7 * 128), (VOCAB, VOCAB)),
                   ((0, 0), (0, VPAD - VOCAB)))
    lse = _lse(table)
    out_t, tgt_part, hist_part = _sc_gather()(idx_f, tgt_f, table, tail)
    logits = jnp.transpose(out_t, (0, 2, 1))
    hist = jnp.sum(hist_part.reshape(NW, VPAD), axis=0)[:VOCAB]
    loss = (jnp.dot(hist, lse) - jnp.sum(tgt_part)) / float(N_TOK)
    return (logits, loss)
